# baseline (speedup 1.0000x reference)
import functools

import jax
import jax.numpy as jnp
from jax import lax
from jax.experimental import pallas as pl
from jax.experimental.pallas import tpu as pltpu
from jax.experimental.pallas import tpu_sc as plsc

_N = 10000
_E = 320000
_D = 128
_L = 3
_NRBF = 32
_RBF_MAX = 10.0
_RES_SCALE = 1000.0
_PW = 16
_TW = _D + _PW

_NC = 2
_NS = 16
_NW = _NC * _NS
_EPW = _E // _NW
_CHUNK = 200
_NLOOP = _EPW // _CHUNK

_BE_EMB = 4000
_BE = 2000
_BN = 2000

_f32 = jnp.float32


def _silu(x):
    return x / (1.0 + jnp.exp(-x))


def _node_embed_body(nf_ref, p_ref, w_ref, b_ref, ws_ref, wd_ref,
                     h_ref, ts_ref, td_ref):
    h = jnp.dot(nf_ref[...], w_ref[...], preferred_element_type=_f32) + b_ref[...]
    h_ref[...] = h
    p = p_ref[...]
    ts_ref[...] = jnp.concatenate(
        [jnp.dot(h, ws_ref[...], preferred_element_type=_f32), p], axis=1)
    td_ref[...] = jnp.concatenate(
        [jnp.dot(h, wd_ref[...], preferred_element_type=_f32), p], axis=1)


def _node_embed(nf8, pos16, Wn8, bn, w1hs, w1hd):
    return pl.pallas_call(
        _node_embed_body,
        out_shape=[
            jax.ShapeDtypeStruct((_N, _D), _f32),
            jax.ShapeDtypeStruct((_N, _TW), _f32),
            jax.ShapeDtypeStruct((_N, _TW), _f32),
        ],
    )(nf8, pos16, Wn8, bn, w1hs, w1hd)


def _edge_embed_body(ea_ref, we_ref, be_ref, o_ref):
    d = ea_ref[...]
    cen = lax.broadcasted_iota(jnp.int32, (1, _NRBF), 1).astype(_f32) * (
        _RBF_MAX / (_NRBF - 1))
    gamma = 1.0 / ((_RBF_MAX / _NRBF) ** 2)
    r = jnp.exp(-gamma * (d - cen) ** 2)
    o_ref[...] = jnp.dot(r, we_ref[...], preferred_element_type=_f32) + be_ref[...]


def _edge_embed(edge_attr, We, be):
    nblk = _E // _BE_EMB
    return pl.pallas_call(
        _edge_embed_body,
        grid=(nblk,),
        in_specs=[
            pl.BlockSpec((_BE_EMB, 1), lambda i: (i, 0)),
            pl.BlockSpec((_NRBF, _D), lambda i: (0, 0)),
            pl.BlockSpec((1, _D), lambda i: (0, 0)),
        ],
        out_specs=pl.BlockSpec((_BE_EMB, _D), lambda i: (i, 0)),
        out_shape=jax.ShapeDtypeStruct((_E, _D), _f32),
    )(edge_attr, We, be)


def _edge_mlp_body(gs_ref, gd_ref, e_ref,
                   w1e_ref, w1d2_ref, b1_ref,
                   w2_ref, b2_ref, wx1_ref, bx1_ref, wx2_ref, bx2_ref,
                   mv_ref):
    gs = gs_ref[...]
    gd = gd_ref[...]
    rel = gd[:, _D:] - gs[:, _D:]
    d2 = jnp.sum(rel * rel, axis=1, keepdims=True)
    z = (gd[:, :_D] + gs[:, :_D]
         + jnp.dot(e_ref[...], w1e_ref[...], preferred_element_type=_f32)
         + d2 * w1d2_ref[...] + b1_ref[...])
    m1 = _silu(z)
    m = _silu(jnp.dot(m1, w2_ref[...], preferred_element_type=_f32) + b2_ref[...])
    t = _silu(jnp.dot(m, wx1_ref[...], preferred_element_type=_f32) + bx1_ref[...])
    w = jnp.sum(t * wx2_ref[...], axis=1, keepdims=True) + bx2_ref[...]
    ones_col = (lax.broadcasted_iota(jnp.int32, (1, _PW), 1) == 3).astype(_f32)
    mv_ref[...] = jnp.concatenate([m, rel * w + ones_col], axis=1)


def _edge_mlp(gs, gd, e, w1e, w1d2, b1, w2, b2, wx1, bx1, wx2, bx2):
    nblk = _E // _BE
    row = lambda i: (i, 0)
    full = lambda i: (0, 0)
    return pl.pallas_call(
        _edge_mlp_body,
        grid=(nblk,),
        in_specs=[
            pl.BlockSpec((_BE, _TW), row),
            pl.BlockSpec((_BE, _TW), row),
            pl.BlockSpec((_BE, _D), row),
            pl.BlockSpec((_D, _D), full),
            pl.BlockSpec((1, _D), full),
            pl.BlockSpec((1, _D), full),
            pl.BlockSpec((_D, _D), full),
            pl.BlockSpec((1, _D), full),
            pl.BlockSpec((_D, _D), full),
            pl.BlockSpec((1, _D), full),
            pl.BlockSpec((1, _D), full),
            pl.BlockSpec((1, 1), full),
        ],
        out_specs=pl.BlockSpec((_BE, _TW), row),
        out_shape=jax.ShapeDtypeStruct((_E, _TW), _f32),
    )(gs, gd, e, w1e, w1d2, b1, w2, b2, wx1, bx1, wx2, bx2)


def _make_node_update_body(emit_tables):
    def body(*refs):
        if emit_tables:
            (h_ref, p_ref, d0_ref, d1_ref,
             wh1h_ref, wh1a_ref, bh1_ref, wh2_ref, bh2_ref, g_ref, b_ref,
             ws_ref, wd_ref,
             ho_ref, po_ref, ts_ref, td_ref) = refs
        else:
            (h_ref, p_ref, d0_ref, d1_ref,
             wh1h_ref, wh1a_ref, bh1_ref, wh2_ref, bh2_ref, g_ref, b_ref,
             ho_ref, po_ref) = refs
        h = h_ref[...]
        d0 = d0_ref[...]
        d1 = d1_ref[...]
        agg = d0[:, :_D] + d1[:, :_D]
        crd = d0[:, _D:] + d1[:, _D:]
        deg = crd[:, 3:4]
        posmask = (lax.broadcasted_iota(jnp.int32, (1, _PW), 1) < 3).astype(_f32)
        pnew = p_ref[...] + crd * posmask / (deg + 1.0)
        po_ref[...] = pnew
        u = _silu(jnp.dot(h, wh1h_ref[...], preferred_element_type=_f32)
                  + jnp.dot(agg, wh1a_ref[...], preferred_element_type=_f32)
                  + bh1_ref[...])
        h2 = h + jnp.dot(u, wh2_ref[...], preferred_element_type=_f32) + bh2_ref[...]
        mu = jnp.mean(h2, axis=1, keepdims=True)
        dc = h2 - mu
        var = jnp.mean(dc * dc, axis=1, keepdims=True)
        hn = dc * lax.rsqrt(var + 1e-5) * g_ref[...] + b_ref[...]
        ho_ref[...] = hn
        if emit_tables:
            ts_ref[...] = jnp.concatenate(
                [jnp.dot(hn, ws_ref[...], preferred_element_type=_f32), pnew],
                axis=1)
            td_ref[...] = jnp.concatenate(
                [jnp.dot(hn, wd_ref[...], preferred_element_type=_f32), pnew],
                axis=1)
    return body


def _node_update(h, pos16, d0, d1, wh1h, wh1a, bh1, wh2, bh2, g, b,
                 ws=None, wd=None):
    emit_tables = ws is not None
    nblk = _N // _BN
    row = lambda i: (i, 0)
    full = lambda i: (0, 0)
    in_specs = [
        pl.BlockSpec((_BN, _D), row),
        pl.BlockSpec((_BN, _PW), row),
        pl.BlockSpec((_BN, _TW), row),
        pl.BlockSpec((_BN, _TW), row),
        pl.BlockSpec((_D, _D), full),
        pl.BlockSpec((_D, _D), full),
        pl.BlockSpec((1, _D), full),
        pl.BlockSpec((_D, _D), full),
        pl.BlockSpec((1, _D), full),
        pl.BlockSpec((1, _D), full),
        pl.BlockSpec((1, _D), full),
    ]
    out_specs = [
        pl.BlockSpec((_BN, _D), row),
        pl.BlockSpec((_BN, _PW), row),
    ]
    out_shape = [
        jax.ShapeDtypeStruct((_N, _D), _f32),
        jax.ShapeDtypeStruct((_N, _PW), _f32),
    ]
    args = [h, pos16, d0, d1, wh1h, wh1a, bh1, wh2, bh2, g, b]
    if emit_tables:
        in_specs += [pl.BlockSpec((_D, _D), full), pl.BlockSpec((_D, _D), full)]
        out_specs += [pl.BlockSpec((_BN, _TW), row), pl.BlockSpec((_BN, _TW), row)]
        out_shape += [jax.ShapeDtypeStruct((_N, _TW), _f32),
                      jax.ShapeDtypeStruct((_N, _TW), _f32)]
        args += [ws, wd]
    return pl.pallas_call(
        _make_node_update_body(emit_tables),
        grid=(nblk,),
        in_specs=in_specs,
        out_specs=out_specs,
        out_shape=out_shape,
    )(*args)


def _sc_gather(ts, td, src, dst):
    mesh = plsc.VectorSubcoreMesh(core_axis_name="c", subcore_axis_name="s")

    @functools.partial(
        pl.kernel,
        mesh=mesh,
        out_type=[
            jax.ShapeDtypeStruct((_E, _TW), _f32),
            jax.ShapeDtypeStruct((_E, _TW), _f32),
        ],
        scratch_types=[
            pltpu.VMEM((2, 2, _CHUNK), jnp.int32),
            pltpu.VMEM((2, _CHUNK, _TW), _f32),
            pltpu.VMEM((2, _CHUNK, _TW), _f32),
            pltpu.SemaphoreType.DMA,
            pltpu.SemaphoreType.DMA,
            pltpu.SemaphoreType.DMA,
        ],
        compiler_params=pltpu.CompilerParams(use_tc_tiling_on_sc=False),
    )
    def k(ts_hbm, td_hbm, src_hbm, dst_hbm,
          gs_hbm, gd_hbm,
          idx2, bs2, bd2, semI, semG, semW):
        c = lax.axis_index("c")
        s = lax.axis_index("s")
        base = (s * _NC + c) * _EPW

        def start_idx(i, p):
            off = pl.multiple_of(base + i * _CHUNK, 8)
            pltpu.async_copy(src_hbm.at[pl.ds(off, _CHUNK)], idx2.at[p, 0], semI)
            pltpu.async_copy(dst_hbm.at[pl.ds(off, _CHUNK)], idx2.at[p, 1], semI)

        def wait_idx(p):
            pltpu.make_async_copy(src_hbm.at[pl.ds(0, _CHUNK)],
                                  idx2.at[p, 0], semI).wait()
            pltpu.make_async_copy(dst_hbm.at[pl.ds(0, _CHUNK)],
                                  idx2.at[p, 1], semI).wait()

        def start_gather(p):
            pltpu.async_copy(ts_hbm.at[idx2.at[p, 0]], bs2.at[p], semG)
            pltpu.async_copy(td_hbm.at[idx2.at[p, 1]], bd2.at[p], semG)

        def wait_gather(p):
            pltpu.make_async_copy(ts_hbm.at[idx2.at[p, 0]],
                                  bs2.at[p], semG).wait()
            pltpu.make_async_copy(td_hbm.at[idx2.at[p, 1]],
                                  bd2.at[p], semG).wait()

        def start_wb(i, p):
            off = pl.multiple_of(base + i * _CHUNK, 8)
            pltpu.async_copy(bs2.at[p], gs_hbm.at[pl.ds(off, _CHUNK)], semW)
            pltpu.async_copy(bd2.at[p], gd_hbm.at[pl.ds(off, _CHUNK)], semW)

        def wait_wb(p):
            pltpu.make_async_copy(bs2.at[p], gs_hbm.at[pl.ds(0, _CHUNK)],
                                  semW).wait()
            pltpu.make_async_copy(bd2.at[p], gd_hbm.at[pl.ds(0, _CHUNK)],
                                  semW).wait()

        start_idx(0, 0)

        def body(i, carry):
            p = lax.rem(i, 2)
            q = 1 - p

            @pl.when(i >= 2)
            def _():
                wait_wb(p)

            @pl.when(i >= 1)
            def _():
                wait_gather(q)
                start_wb(i - 1, q)

            wait_idx(p)
            start_gather(p)

            @pl.when(i + 1 < _NLOOP)
            def _():
                start_idx(i + 1, q)

            return carry

        lax.fori_loop(0, _NLOOP, body, 0)

        last = (_NLOOP - 1) % 2
        wait_gather(last)
        start_wb(_NLOOP - 1, last)
        wait_wb(1 - last)
        wait_wb(last)

    return k(ts, td, src, dst)


def _sc_scatter(mv, dst, z144):
    mesh = plsc.VectorSubcoreMesh(core_axis_name="c", subcore_axis_name="s")

    @functools.partial(
        pl.kernel,
        mesh=mesh,
        out_type=jax.ShapeDtypeStruct((_NC, _N, _TW), _f32),
        scratch_types=[
            pltpu.VMEM((1, _CHUNK), jnp.int32),
            pltpu.VMEM((_CHUNK, _TW), _f32),
            pltpu.VMEM_SHARED((_N, _TW), _f32),
            pltpu.SemaphoreType.DMA,
            pltpu.SemaphoreType.DMA,
        ],
        compiler_params=pltpu.CompilerParams(use_tc_tiling_on_sc=False),
    )
    def k(mv_hbm, dst_hbm, z_hbm, out_hbm, idx, vb, sh, semL, semA):
        c = lax.axis_index("c")
        s = lax.axis_index("s")
        base = (c * _NS + s) * _EPW

        @pl.when(s == 0)
        def _():
            pltpu.sync_copy(z_hbm, sh)

        plsc.subcore_barrier()

        def start_loads(i):
            off = pl.multiple_of(base + i * _CHUNK, 8)
            pltpu.async_copy(dst_hbm.at[pl.ds(off, _CHUNK)], idx.at[0], semL)
            pltpu.async_copy(mv_hbm.at[pl.ds(off, _CHUNK)], vb, semL)

        def wait_loads():
            pltpu.make_async_copy(dst_hbm.at[pl.ds(0, _CHUNK)],
                                  idx.at[0], semL).wait()
            pltpu.make_async_copy(mv_hbm.at[pl.ds(0, _CHUNK)], vb, semL).wait()

        def body(i, carry):
            wait_loads()
            pltpu.async_copy(vb, sh.at[idx.at[0]], semA, add=True).wait()

            @pl.when(i + 1 < _NLOOP)
            def _():
                start_loads(i + 1)

            return carry

        start_loads(0)
        lax.fori_loop(0, _NLOOP, body, 0)

        plsc.subcore_barrier()

        @pl.when(s == 0)
        def _():
            pltpu.sync_copy(sh, out_hbm.at[c])

    return k(mv, dst, z144)


def kernel(node_feat, edge_attr, pos, Wn, bn, We, be, We1, be1, We2, be2,
           Wx1, bx1, Wx2, bx2, Wh1, bh1, Wh2, bh2, ln_g, ln_b, edge_index):
    src = edge_index[0]
    dst = edge_index[1]

    nf8 = jnp.concatenate(
        [node_feat[:, :6], node_feat[:, 6:7] / _RES_SCALE,
         jnp.zeros((_N, 1), _f32)], axis=1)
    Wn8 = jnp.concatenate([Wn, jnp.zeros((1, _D), _f32)], axis=0)
    pos16 = jnp.concatenate([pos, jnp.zeros((_N, _PW - 3), _f32)], axis=1)
    z144 = jnp.zeros((_N, _TW), _f32)

    h, ts, td = _node_embed(nf8, pos16, Wn8, bn.reshape(1, _D),
                            We1[0, _D:2 * _D], We1[0, 0:_D])
    e = _edge_embed(edge_attr, We, be.reshape(1, _D))

    for l in range(_L):
        w1d2 = We1[l, 2 * _D:2 * _D + 1]
        w1e = We1[l, 2 * _D + 1:]

        gs, gd = _sc_gather(ts, td, src, dst)
        mv = _edge_mlp(gs, gd, e,
                       w1e, w1d2, be1[l].reshape(1, _D),
                       We2[l], be2[l].reshape(1, _D),
                       Wx1[l], bx1[l].reshape(1, _D),
                       Wx2[l].reshape(1, _D), bx2[l].reshape(1, 1))
        dump = _sc_scatter(mv, dst, z144)
        if l + 1 < _L:
            h, pos16, ts, td = _node_update(
                h, pos16, dump[0], dump[1],
                Wh1[l, :_D], Wh1[l, _D:], bh1[l].reshape(1, _D),
                Wh2[l], bh2[l].reshape(1, _D),
                ln_g[l].reshape(1, _D), ln_b[l].reshape(1, _D),
                We1[l + 1, _D:2 * _D], We1[l + 1, 0:_D])
        else:
            h, pos16 = _node_update(
                h, pos16, dump[0], dump[1],
                Wh1[l, :_D], Wh1[l, _D:], bh1[l].reshape(1, _D),
                Wh2[l], bh2[l].reshape(1, _D),
                ln_g[l].reshape(1, _D), ln_b[l].reshape(1, _D))

    return h, pos16[:, :3]


# speedup vs baseline: 1.5159x; 1.5159x over previous
import functools

import jax
import jax.numpy as jnp
from jax import lax
from jax.experimental import pallas as pl
from jax.experimental.pallas import tpu as pltpu
from jax.experimental.pallas import tpu_sc as plsc

_N = 10000
_E = 320000
_D = 128
_L = 3
_NRBF = 32
_RBF_MAX = 10.0
_RES_SCALE = 1000.0
_PW = 16

_NC = 2
_NS = 16
_NW = _NC * _NS
_EPW = _E // _NW
_CHUNK = 200
_NLOOP = _EPW // _CHUNK

_BE_EMB = 4000
_BE = 2000
_BN = 2000

_f32 = jnp.float32


def _silu(x):
    return x / (1.0 + jnp.exp(-x))


def _node_embed_body(nf_ref, w_ref, b_ref, ws_ref, wd_ref,
                     h_ref, ts_ref, td_ref):
    h = jnp.dot(nf_ref[...], w_ref[...], preferred_element_type=_f32) + b_ref[...]
    h_ref[...] = h
    ts_ref[...] = jnp.dot(h, ws_ref[...], preferred_element_type=_f32)
    td_ref[...] = jnp.dot(h, wd_ref[...], preferred_element_type=_f32)


def _node_embed(nf8, Wn8, bn, w1hs, w1hd):
    return pl.pallas_call(
        _node_embed_body,
        out_shape=[
            jax.ShapeDtypeStruct((_N, _D), _f32),
            jax.ShapeDtypeStruct((_N, _D), _f32),
            jax.ShapeDtypeStruct((_N, _D), _f32),
        ],
    )(nf8, Wn8, bn, w1hs, w1hd)


def _edge_embed_body(ea_ref, we_ref, be_ref, o_ref):
    d = ea_ref[...]
    cen = lax.broadcasted_iota(jnp.int32, (1, _NRBF), 1).astype(_f32) * (
        _RBF_MAX / (_NRBF - 1))
    gamma = 1.0 / ((_RBF_MAX / _NRBF) ** 2)
    r = jnp.exp(-gamma * (d - cen) ** 2)
    o_ref[...] = jnp.dot(r, we_ref[...], preferred_element_type=_f32) + be_ref[...]


def _edge_embed(edge_attr, We, be):
    nblk = _E // _BE_EMB
    return pl.pallas_call(
        _edge_embed_body,
        grid=(nblk,),
        in_specs=[
            pl.BlockSpec((_BE_EMB, 1), lambda i: (i, 0)),
            pl.BlockSpec((_NRBF, _D), lambda i: (0, 0)),
            pl.BlockSpec((1, _D), lambda i: (0, 0)),
        ],
        out_specs=pl.BlockSpec((_BE_EMB, _D), lambda i: (i, 0)),
        out_shape=jax.ShapeDtypeStruct((_E, _D), _f32),
    )(edge_attr, We, be)


def _edge_mlp_body(gs_ref, gd_ref, e_ref, ps_ref, pd_ref,
                   w1e_ref, w1d2_ref, b1_ref,
                   w2_ref, b2_ref, wx1_ref, bx1_ref, wx2_ref, bx2_ref,
                   m_ref, wrel_ref):
    rel = pd_ref[...] - ps_ref[...]
    d2 = jnp.sum(rel * rel, axis=1, keepdims=True)
    z = (gd_ref[...] + gs_ref[...]
         + jnp.dot(e_ref[...], w1e_ref[...], preferred_element_type=_f32)
         + d2 * w1d2_ref[...] + b1_ref[...])
    m1 = _silu(z)
    m = _silu(jnp.dot(m1, w2_ref[...], preferred_element_type=_f32) + b2_ref[...])
    t = _silu(jnp.dot(m, wx1_ref[...], preferred_element_type=_f32) + bx1_ref[...])
    w = jnp.sum(t * wx2_ref[...], axis=1, keepdims=True) + bx2_ref[...]
    m_ref[...] = m
    ones_col = (lax.broadcasted_iota(jnp.int32, (1, _PW), 1) == 3).astype(_f32)
    wrel_ref[...] = rel * w + ones_col


def _edge_mlp(gs, gd, e, ps, pd, w1e, w1d2, b1, w2, b2, wx1, bx1, wx2, bx2):
    nblk = _E // _BE
    row = lambda i: (i, 0)
    full = lambda i: (0, 0)
    return pl.pallas_call(
        _edge_mlp_body,
        grid=(nblk,),
        in_specs=[
            pl.BlockSpec((_BE, _D), row),
            pl.BlockSpec((_BE, _D), row),
            pl.BlockSpec((_BE, _D), row),
            pl.BlockSpec((_BE, _PW), row),
            pl.BlockSpec((_BE, _PW), row),
            pl.BlockSpec((_D, _D), full),
            pl.BlockSpec((1, _D), full),
            pl.BlockSpec((1, _D), full),
            pl.BlockSpec((_D, _D), full),
            pl.BlockSpec((1, _D), full),
            pl.BlockSpec((_D, _D), full),
            pl.BlockSpec((1, _D), full),
            pl.BlockSpec((1, _D), full),
            pl.BlockSpec((1, 1), full),
        ],
        out_specs=[
            pl.BlockSpec((_BE, _D), row),
            pl.BlockSpec((_BE, _PW), row),
        ],
        out_shape=[
            jax.ShapeDtypeStruct((_E, _D), _f32),
            jax.ShapeDtypeStruct((_E, _PW), _f32),
        ],
    )(gs, gd, e, ps, pd, w1e, w1d2, b1, w2, b2, wx1, bx1, wx2, bx2)


def _make_node_update_body(emit_tables):
    def body(*refs):
        if emit_tables:
            (h_ref, p_ref, a0_ref, a1_ref, c0_ref, c1_ref,
             wh1h_ref, wh1a_ref, bh1_ref, wh2_ref, bh2_ref, g_ref, b_ref,
             ws_ref, wd_ref,
             ho_ref, po_ref, ts_ref, td_ref) = refs
        else:
            (h_ref, p_ref, a0_ref, a1_ref, c0_ref, c1_ref,
             wh1h_ref, wh1a_ref, bh1_ref, wh2_ref, bh2_ref, g_ref, b_ref,
             ho_ref, po_ref) = refs
        h = h_ref[...]
        agg = a0_ref[...] + a1_ref[...]
        crd = c0_ref[...] + c1_ref[...]
        deg = crd[:, 3:4]
        posmask = (lax.broadcasted_iota(jnp.int32, (1, _PW), 1) < 3).astype(_f32)
        po_ref[...] = p_ref[...] + crd * posmask / (deg + 1.0)
        u = _silu(jnp.dot(h, wh1h_ref[...], preferred_element_type=_f32)
                  + jnp.dot(agg, wh1a_ref[...], preferred_element_type=_f32)
                  + bh1_ref[...])
        h2 = h + jnp.dot(u, wh2_ref[...], preferred_element_type=_f32) + bh2_ref[...]
        mu = jnp.mean(h2, axis=1, keepdims=True)
        dc = h2 - mu
        var = jnp.mean(dc * dc, axis=1, keepdims=True)
        hn = dc * lax.rsqrt(var + 1e-5) * g_ref[...] + b_ref[...]
        ho_ref[...] = hn
        if emit_tables:
            ts_ref[...] = jnp.dot(hn, ws_ref[...], preferred_element_type=_f32)
            td_ref[...] = jnp.dot(hn, wd_ref[...], preferred_element_type=_f32)
    return body


def _node_update(h, pos16, a0, a1, c0, c1, wh1h, wh1a, bh1, wh2, bh2, g, b,
                 ws=None, wd=None):
    emit_tables = ws is not None
    nblk = _N // _BN
    row = lambda i: (i, 0)
    full = lambda i: (0, 0)
    in_specs = [
        pl.BlockSpec((_BN, _D), row),
        pl.BlockSpec((_BN, _PW), row),
        pl.BlockSpec((_BN, _D), row),
        pl.BlockSpec((_BN, _D), row),
        pl.BlockSpec((_BN, _PW), row),
        pl.BlockSpec((_BN, _PW), row),
        pl.BlockSpec((_D, _D), full),
        pl.BlockSpec((_D, _D), full),
        pl.BlockSpec((1, _D), full),
        pl.BlockSpec((_D, _D), full),
        pl.BlockSpec((1, _D), full),
        pl.BlockSpec((1, _D), full),
        pl.BlockSpec((1, _D), full),
    ]
    out_specs = [
        pl.BlockSpec((_BN, _D), row),
        pl.BlockSpec((_BN, _PW), row),
    ]
    out_shape = [
        jax.ShapeDtypeStruct((_N, _D), _f32),
        jax.ShapeDtypeStruct((_N, _PW), _f32),
    ]
    args = [h, pos16, a0, a1, c0, c1, wh1h, wh1a, bh1, wh2, bh2, g, b]
    if emit_tables:
        in_specs += [pl.BlockSpec((_D, _D), full), pl.BlockSpec((_D, _D), full)]
        out_specs += [pl.BlockSpec((_BN, _D), row), pl.BlockSpec((_BN, _D), row)]
        out_shape += [jax.ShapeDtypeStruct((_N, _D), _f32),
                      jax.ShapeDtypeStruct((_N, _D), _f32)]
        args += [ws, wd]
    return pl.pallas_call(
        _make_node_update_body(emit_tables),
        grid=(nblk,),
        in_specs=in_specs,
        out_specs=out_specs,
        out_shape=out_shape,
    )(*args)


_RC = 128
_NROW = _E // _RC
_RPW = _NROW // _NW
_RREM = _NROW % _NW


def _sc_gather128(ts, td, src, dst):
    mesh = plsc.VectorSubcoreMesh(core_axis_name="c", subcore_axis_name="s")

    @functools.partial(
        pl.kernel,
        mesh=mesh,
        out_type=[
            jax.ShapeDtypeStruct((_E, _D), _f32),
            jax.ShapeDtypeStruct((_E, _D), _f32),
        ],
        scratch_types=[
            pltpu.VMEM((2, 2, _RC), jnp.int32),
            pltpu.VMEM((2, _RC, _D), _f32),
            pltpu.VMEM((2, _RC, _D), _f32),
            pltpu.SemaphoreType.DMA,
            pltpu.SemaphoreType.DMA,
            pltpu.SemaphoreType.DMA,
        ],
        compiler_params=pltpu.CompilerParams(use_tc_tiling_on_sc=True),
    )
    def k(ts_hbm, td_hbm, src_hbm, dst_hbm,
          gs_hbm, gd_hbm,
          idx2, bs2, bd2, semI, semG, semW):
        c = lax.axis_index("c")
        s = lax.axis_index("s")
        w = s * _NC + c
        nr = _RPW + jnp.where(w < _RREM, 1, 0)
        base_row = _RPW * w + jnp.minimum(w, _RREM)

        def start_idx(i, p):
            off = pl.multiple_of((base_row + i) * _RC, _RC)
            pltpu.async_copy(src_hbm.at[pl.ds(off, _RC)], idx2.at[p, 0], semI)
            pltpu.async_copy(dst_hbm.at[pl.ds(off, _RC)], idx2.at[p, 1], semI)

        def wait_idx(p):
            pltpu.make_async_copy(src_hbm.at[pl.ds(0, _RC)],
                                  idx2.at[p, 0], semI).wait()
            pltpu.make_async_copy(dst_hbm.at[pl.ds(0, _RC)],
                                  idx2.at[p, 1], semI).wait()

        def start_gather(p):
            pltpu.async_copy(ts_hbm.at[idx2.at[p, 0]], bs2.at[p], semG)
            pltpu.async_copy(td_hbm.at[idx2.at[p, 1]], bd2.at[p], semG)

        def wait_gather(p):
            pltpu.make_async_copy(ts_hbm.at[idx2.at[p, 0]],
                                  bs2.at[p], semG).wait()
            pltpu.make_async_copy(td_hbm.at[idx2.at[p, 1]],
                                  bd2.at[p], semG).wait()

        def start_wb(i, p):
            off = pl.multiple_of((base_row + i) * _RC, _RC)
            pltpu.async_copy(bs2.at[p], gs_hbm.at[pl.ds(off, _RC)], semW)
            pltpu.async_copy(bd2.at[p], gd_hbm.at[pl.ds(off, _RC)], semW)

        def wait_wb(p):
            pltpu.make_async_copy(bs2.at[p], gs_hbm.at[pl.ds(0, _RC)],
                                  semW).wait()
            pltpu.make_async_copy(bd2.at[p], gd_hbm.at[pl.ds(0, _RC)],
                                  semW).wait()

        start_idx(0, 0)

        def body(i, carry):
            p = lax.rem(i, 2)
            q = 1 - p

            @pl.when(i >= 2)
            def _():
                wait_wb(p)

            @pl.when(i >= 1)
            def _():
                wait_gather(q)
                start_wb(i - 1, q)

            wait_idx(p)
            start_gather(p)

            @pl.when(i + 1 < nr)
            def _():
                start_idx(i + 1, q)

            return carry

        lax.fori_loop(0, nr, body, 0)

        last = lax.rem(nr - 1, 2)
        wait_gather(last)
        start_wb(nr - 1, last)
        wait_wb(1 - last)
        wait_wb(last)

    return k(ts, td, src, dst)


def _sc_scatter128(vals, dst, zeros):
    mesh = plsc.VectorSubcoreMesh(core_axis_name="c", subcore_axis_name="s")

    @functools.partial(
        pl.kernel,
        mesh=mesh,
        out_type=jax.ShapeDtypeStruct((_NC, _N, _D), _f32),
        scratch_types=[
            pltpu.VMEM((2, 1, _RC), jnp.int32),
            pltpu.VMEM((2, _RC, _D), _f32),
            pltpu.VMEM_SHARED((_N, _D), _f32),
            pltpu.SemaphoreType.DMA,
            pltpu.SemaphoreType.DMA,
        ],
        compiler_params=pltpu.CompilerParams(use_tc_tiling_on_sc=True),
    )
    def k(v_hbm, dst_hbm, z_hbm, out_hbm, idx2, vb2, sh, semL, semA):
        c = lax.axis_index("c")
        s = lax.axis_index("s")
        w = c * _NS + s
        nr = _RPW + jnp.where(w < _RREM, 1, 0)
        base_row = _RPW * w + jnp.minimum(w, _RREM)

        @pl.when(s == 0)
        def _():
            pltpu.sync_copy(z_hbm, sh)

        plsc.subcore_barrier()

        def start_loads(i, p):
            off = pl.multiple_of((base_row + i) * _RC, _RC)
            pltpu.async_copy(dst_hbm.at[pl.ds(off, _RC)], idx2.at[p, 0], semL)
            pltpu.async_copy(v_hbm.at[pl.ds(off, _RC)], vb2.at[p], semL)

        def wait_loads(p):
            pltpu.make_async_copy(dst_hbm.at[pl.ds(0, _RC)],
                                  idx2.at[p, 0], semL).wait()
            pltpu.make_async_copy(v_hbm.at[pl.ds(0, _RC)],
                                  vb2.at[p], semL).wait()

        def wait_add(p):
            pltpu.make_async_copy(vb2.at[p], sh.at[idx2.at[p, 0]],
                                  semA).wait()

        start_loads(0, 0)

        def body(i, carry):
            p = lax.rem(i, 2)
            q = 1 - p

            @pl.when(i >= 1)
            def _():
                wait_add(q)

            wait_loads(p)
            pltpu.async_copy(vb2.at[p], sh.at[idx2.at[p, 0]], semA, add=True)

            @pl.when(i + 1 < nr)
            def _():
                start_loads(i + 1, q)

            return carry

        lax.fori_loop(0, nr, body, 0)
        wait_add(lax.rem(nr - 1, 2))

        plsc.subcore_barrier()

        @pl.when(s == 0)
        def _():
            pltpu.sync_copy(sh, out_hbm.at[c])

    return k(vals, dst, zeros)


def _sc_gather(ts, td, src, dst, width, tc_tiling):
    mesh = plsc.VectorSubcoreMesh(core_axis_name="c", subcore_axis_name="s")

    @functools.partial(
        pl.kernel,
        mesh=mesh,
        out_type=[
            jax.ShapeDtypeStruct((_E, width), _f32),
            jax.ShapeDtypeStruct((_E, width), _f32),
        ],
        scratch_types=[
            pltpu.VMEM((2, 2, _CHUNK), jnp.int32),
            pltpu.VMEM((2, _CHUNK, width), _f32),
            pltpu.VMEM((2, _CHUNK, width), _f32),
            pltpu.SemaphoreType.DMA,
            pltpu.SemaphoreType.DMA,
            pltpu.SemaphoreType.DMA,
        ],
        compiler_params=pltpu.CompilerParams(use_tc_tiling_on_sc=tc_tiling),
    )
    def k(ts_hbm, td_hbm, src_hbm, dst_hbm,
          gs_hbm, gd_hbm,
          idx2, bs2, bd2, semI, semG, semW):
        c = lax.axis_index("c")
        s = lax.axis_index("s")
        base = (s * _NC + c) * _EPW

        def start_idx(i, p):
            off = pl.multiple_of(base + i * _CHUNK, 8)
            pltpu.async_copy(src_hbm.at[pl.ds(off, _CHUNK)], idx2.at[p, 0], semI)
            pltpu.async_copy(dst_hbm.at[pl.ds(off, _CHUNK)], idx2.at[p, 1], semI)

        def wait_idx(p):
            pltpu.make_async_copy(src_hbm.at[pl.ds(0, _CHUNK)],
                                  idx2.at[p, 0], semI).wait()
            pltpu.make_async_copy(dst_hbm.at[pl.ds(0, _CHUNK)],
                                  idx2.at[p, 1], semI).wait()

        def start_gather(p):
            pltpu.async_copy(ts_hbm.at[idx2.at[p, 0]], bs2.at[p], semG)
            pltpu.async_copy(td_hbm.at[idx2.at[p, 1]], bd2.at[p], semG)

        def wait_gather(p):
            pltpu.make_async_copy(ts_hbm.at[idx2.at[p, 0]],
                                  bs2.at[p], semG).wait()
            pltpu.make_async_copy(td_hbm.at[idx2.at[p, 1]],
                                  bd2.at[p], semG).wait()

        def start_wb(i, p):
            off = pl.multiple_of(base + i * _CHUNK, 8)
            pltpu.async_copy(bs2.at[p], gs_hbm.at[pl.ds(off, _CHUNK)], semW)
            pltpu.async_copy(bd2.at[p], gd_hbm.at[pl.ds(off, _CHUNK)], semW)

        def wait_wb(p):
            pltpu.make_async_copy(bs2.at[p], gs_hbm.at[pl.ds(0, _CHUNK)],
                                  semW).wait()
            pltpu.make_async_copy(bd2.at[p], gd_hbm.at[pl.ds(0, _CHUNK)],
                                  semW).wait()

        start_idx(0, 0)

        def body(i, carry):
            p = lax.rem(i, 2)
            q = 1 - p

            @pl.when(i >= 2)
            def _():
                wait_wb(p)

            @pl.when(i >= 1)
            def _():
                wait_gather(q)
                start_wb(i - 1, q)

            wait_idx(p)
            start_gather(p)

            @pl.when(i + 1 < _NLOOP)
            def _():
                start_idx(i + 1, q)

            return carry

        lax.fori_loop(0, _NLOOP, body, 0)

        last = (_NLOOP - 1) % 2
        wait_gather(last)
        start_wb(_NLOOP - 1, last)
        wait_wb(1 - last)
        wait_wb(last)

    return k(ts, td, src, dst)


def _sc_scatter(vals, dst, zeros, width, tc_tiling):
    mesh = plsc.VectorSubcoreMesh(core_axis_name="c", subcore_axis_name="s")

    @functools.partial(
        pl.kernel,
        mesh=mesh,
        out_type=jax.ShapeDtypeStruct((_NC, _N, width), _f32),
        scratch_types=[
            pltpu.VMEM((1, _CHUNK), jnp.int32),
            pltpu.VMEM((_CHUNK, width), _f32),
            pltpu.VMEM_SHARED((_N, width), _f32),
            pltpu.SemaphoreType.DMA,
            pltpu.SemaphoreType.DMA,
        ],
        compiler_params=pltpu.CompilerParams(use_tc_tiling_on_sc=tc_tiling),
    )
    def k(v_hbm, dst_hbm, z_hbm, out_hbm, idx, vb, sh, semL, semA):
        c = lax.axis_index("c")
        s = lax.axis_index("s")
        base = (c * _NS + s) * _EPW

        @pl.when(s == 0)
        def _():
            pltpu.sync_copy(z_hbm, sh)

        plsc.subcore_barrier()

        def start_loads(i):
            off = pl.multiple_of(base + i * _CHUNK, 8)
            pltpu.async_copy(dst_hbm.at[pl.ds(off, _CHUNK)], idx.at[0], semL)
            pltpu.async_copy(v_hbm.at[pl.ds(off, _CHUNK)], vb, semL)

        def wait_loads():
            pltpu.make_async_copy(dst_hbm.at[pl.ds(0, _CHUNK)],
                                  idx.at[0], semL).wait()
            pltpu.make_async_copy(v_hbm.at[pl.ds(0, _CHUNK)], vb, semL).wait()

        def body(i, carry):
            wait_loads()
            pltpu.async_copy(vb, sh.at[idx.at[0]], semA, add=True).wait()

            @pl.when(i + 1 < _NLOOP)
            def _():
                start_loads(i + 1)

            return carry

        start_loads(0)
        lax.fori_loop(0, _NLOOP, body, 0)

        plsc.subcore_barrier()

        @pl.when(s == 0)
        def _():
            pltpu.sync_copy(sh, out_hbm.at[c])

    return k(vals, dst, zeros)


def kernel(node_feat, edge_attr, pos, Wn, bn, We, be, We1, be1, We2, be2,
           Wx1, bx1, Wx2, bx2, Wh1, bh1, Wh2, bh2, ln_g, ln_b, edge_index):
    src = edge_index[0]
    dst = edge_index[1]

    nf8 = jnp.concatenate(
        [node_feat[:, :6], node_feat[:, 6:7] / _RES_SCALE,
         jnp.zeros((_N, 1), _f32)], axis=1)
    Wn8 = jnp.concatenate([Wn, jnp.zeros((1, _D), _f32)], axis=0)
    pos16 = jnp.concatenate([pos, jnp.zeros((_N, _PW - 3), _f32)], axis=1)
    z128 = jnp.zeros((_N, _D), _f32)
    z16 = jnp.zeros((_N, _PW), _f32)

    h, ts, td = _node_embed(nf8, Wn8, bn.reshape(1, _D),
                            We1[0, _D:2 * _D], We1[0, 0:_D])
    e = _edge_embed(edge_attr, We, be.reshape(1, _D))

    for l in range(_L):
        w1d2 = We1[l, 2 * _D:2 * _D + 1]
        w1e = We1[l, 2 * _D + 1:]

        gs, gd = _sc_gather128(ts, td, src, dst)
        ps, pd = _sc_gather(pos16, pos16, src, dst, _PW, False)
        m, wrel = _edge_mlp(gs, gd, e, ps, pd,
                            w1e, w1d2, be1[l].reshape(1, _D),
                            We2[l], be2[l].reshape(1, _D),
                            Wx1[l], bx1[l].reshape(1, _D),
                            Wx2[l].reshape(1, _D), bx2[l].reshape(1, 1))
        aggp = _sc_scatter128(m, dst, z128)
        crdp = _sc_scatter(wrel, dst, z16, _PW, False)
        if l + 1 < _L:
            h, pos16, ts, td = _node_update(
                h, pos16, aggp[0], aggp[1], crdp[0], crdp[1],
                Wh1[l, :_D], Wh1[l, _D:], bh1[l].reshape(1, _D),
                Wh2[l], bh2[l].reshape(1, _D),
                ln_g[l].reshape(1, _D), ln_b[l].reshape(1, _D),
                We1[l + 1, _D:2 * _D], We1[l + 1, 0:_D])
        else:
            h, pos16 = _node_update(
                h, pos16, aggp[0], aggp[1], crdp[0], crdp[1],
                Wh1[l, :_D], Wh1[l, _D:], bh1[l].reshape(1, _D),
                Wh2[l], bh2[l].reshape(1, _D),
                ln_g[l].reshape(1, _D), ln_b[l].reshape(1, _D))

    return h, pos16[:, :3]


# speedup vs baseline: 1.5235x; 1.0050x over previous
import functools

import jax
import jax.numpy as jnp
from jax import lax
from jax.experimental import pallas as pl
from jax.experimental.pallas import tpu as pltpu
from jax.experimental.pallas import tpu_sc as plsc

_N = 10000
_E = 320000
_D = 128
_L = 3
_NRBF = 32
_RBF_MAX = 10.0
_RES_SCALE = 1000.0
_PW = 16

_NC = 2
_NS = 16
_NW = _NC * _NS
_EPW = _E // _NW
_CHUNK = 200
_NLOOP = _EPW // _CHUNK

_BE_EMB = 4000
_BE = 2000
_BN = 2000

_f32 = jnp.float32


def _silu(x):
    return x / (1.0 + jnp.exp(-x))


def _node_embed_body(nf_ref, w_ref, b_ref, ws_ref, wd_ref,
                     h_ref, ts_ref, td_ref):
    h = jnp.dot(nf_ref[...], w_ref[...], preferred_element_type=_f32) + b_ref[...]
    h_ref[...] = h
    ts_ref[...] = jnp.dot(h, ws_ref[...], preferred_element_type=_f32)
    td_ref[...] = jnp.dot(h, wd_ref[...], preferred_element_type=_f32)


def _node_embed(nf8, Wn8, bn, w1hs, w1hd):
    return pl.pallas_call(
        _node_embed_body,
        out_shape=[
            jax.ShapeDtypeStruct((_N, _D), _f32),
            jax.ShapeDtypeStruct((_N, _D), _f32),
            jax.ShapeDtypeStruct((_N, _D), _f32),
        ],
    )(nf8, Wn8, bn, w1hs, w1hd)


def _edge_embed_body(ea_ref, we_ref, be_ref, o_ref):
    d = ea_ref[...]
    cen = lax.broadcasted_iota(jnp.int32, (1, _NRBF), 1).astype(_f32) * (
        _RBF_MAX / (_NRBF - 1))
    gamma = 1.0 / ((_RBF_MAX / _NRBF) ** 2)
    r = jnp.exp(-gamma * (d - cen) ** 2)
    o_ref[...] = jnp.dot(r, we_ref[...], preferred_element_type=_f32) + be_ref[...]


def _edge_embed(edge_attr, We, be):
    nblk = _E // _BE_EMB
    return pl.pallas_call(
        _edge_embed_body,
        grid=(nblk,),
        in_specs=[
            pl.BlockSpec((_BE_EMB, 1), lambda i: (i, 0)),
            pl.BlockSpec((_NRBF, _D), lambda i: (0, 0)),
            pl.BlockSpec((1, _D), lambda i: (0, 0)),
        ],
        out_specs=pl.BlockSpec((_BE_EMB, _D), lambda i: (i, 0)),
        out_shape=jax.ShapeDtypeStruct((_E, _D), _f32),
    )(edge_attr, We, be)


def _edge_mlp_body(gs_ref, gd_ref, e_ref, ps_ref, pd_ref,
                   w1e_ref, w1d2_ref, b1_ref,
                   w2_ref, b2_ref, wx1_ref, bx1_ref, wx2_ref, bx2_ref,
                   m_ref, wrel_ref):
    rel = pd_ref[...] - ps_ref[...]
    d2 = jnp.sum(rel * rel, axis=1, keepdims=True)
    z = (gd_ref[...] + gs_ref[...]
         + jnp.dot(e_ref[...], w1e_ref[...], preferred_element_type=_f32)
         + d2 * w1d2_ref[...] + b1_ref[...])
    m1 = _silu(z)
    m = _silu(jnp.dot(m1, w2_ref[...], preferred_element_type=_f32) + b2_ref[...])
    t = _silu(jnp.dot(m, wx1_ref[...], preferred_element_type=_f32) + bx1_ref[...])
    w = jnp.sum(t * wx2_ref[...], axis=1, keepdims=True) + bx2_ref[...]
    m_ref[...] = m
    ones_col = (lax.broadcasted_iota(jnp.int32, (1, _D), 1) == 3).astype(_f32)
    wrel_ref[...] = rel * w + ones_col


def _edge_mlp(gs, gd, e, ps, pd, w1e, w1d2, b1, w2, b2, wx1, bx1, wx2, bx2):
    nblk = _E // _BE
    row = lambda i: (i, 0)
    full = lambda i: (0, 0)
    return pl.pallas_call(
        _edge_mlp_body,
        grid=(nblk,),
        in_specs=[
            pl.BlockSpec((_BE, _D), row),
            pl.BlockSpec((_BE, _D), row),
            pl.BlockSpec((_BE, _D), row),
            pl.BlockSpec((_BE, _D), row),
            pl.BlockSpec((_BE, _D), row),
            pl.BlockSpec((_D, _D), full),
            pl.BlockSpec((1, _D), full),
            pl.BlockSpec((1, _D), full),
            pl.BlockSpec((_D, _D), full),
            pl.BlockSpec((1, _D), full),
            pl.BlockSpec((_D, _D), full),
            pl.BlockSpec((1, _D), full),
            pl.BlockSpec((1, _D), full),
            pl.BlockSpec((1, 1), full),
        ],
        out_specs=[
            pl.BlockSpec((_BE, _D), row),
            pl.BlockSpec((_BE, _D), row),
        ],
        out_shape=[
            jax.ShapeDtypeStruct((_E, _D), _f32),
            jax.ShapeDtypeStruct((_E, _D), _f32),
        ],
    )(gs, gd, e, ps, pd, w1e, w1d2, b1, w2, b2, wx1, bx1, wx2, bx2)


def _make_node_update_body(emit_tables):
    def body(*refs):
        if emit_tables:
            (h_ref, p_ref, a0_ref, a1_ref, c0_ref, c1_ref,
             wh1h_ref, wh1a_ref, bh1_ref, wh2_ref, bh2_ref, g_ref, b_ref,
             ws_ref, wd_ref,
             ho_ref, po_ref, ts_ref, td_ref) = refs
        else:
            (h_ref, p_ref, a0_ref, a1_ref, c0_ref, c1_ref,
             wh1h_ref, wh1a_ref, bh1_ref, wh2_ref, bh2_ref, g_ref, b_ref,
             ho_ref, po_ref) = refs
        h = h_ref[...]
        agg = a0_ref[...] + a1_ref[...]
        crd = c0_ref[...] + c1_ref[...]
        deg = crd[:, 3:4]
        posmask = (lax.broadcasted_iota(jnp.int32, (1, _D), 1) < 3).astype(_f32)
        po_ref[...] = p_ref[...] + crd * posmask / (deg + 1.0)
        u = _silu(jnp.dot(h, wh1h_ref[...], preferred_element_type=_f32)
                  + jnp.dot(agg, wh1a_ref[...], preferred_element_type=_f32)
                  + bh1_ref[...])
        h2 = h + jnp.dot(u, wh2_ref[...], preferred_element_type=_f32) + bh2_ref[...]
        mu = jnp.mean(h2, axis=1, keepdims=True)
        dc = h2 - mu
        var = jnp.mean(dc * dc, axis=1, keepdims=True)
        hn = dc * lax.rsqrt(var + 1e-5) * g_ref[...] + b_ref[...]
        ho_ref[...] = hn
        if emit_tables:
            ts_ref[...] = jnp.dot(hn, ws_ref[...], preferred_element_type=_f32)
            td_ref[...] = jnp.dot(hn, wd_ref[...], preferred_element_type=_f32)
    return body


def _node_update(h, pos128, a0, a1, c0, c1, wh1h, wh1a, bh1, wh2, bh2, g, b,
                 ws=None, wd=None):
    emit_tables = ws is not None
    nblk = _N // _BN
    row = lambda i: (i, 0)
    full = lambda i: (0, 0)
    in_specs = [
        pl.BlockSpec((_BN, _D), row),
        pl.BlockSpec((_BN, _D), row),
        pl.BlockSpec((_BN, _D), row),
        pl.BlockSpec((_BN, _D), row),
        pl.BlockSpec((_BN, _D), row),
        pl.BlockSpec((_BN, _D), row),
        pl.BlockSpec((_D, _D), full),
        pl.BlockSpec((_D, _D), full),
        pl.BlockSpec((1, _D), full),
        pl.BlockSpec((_D, _D), full),
        pl.BlockSpec((1, _D), full),
        pl.BlockSpec((1, _D), full),
        pl.BlockSpec((1, _D), full),
    ]
    out_specs = [
        pl.BlockSpec((_BN, _D), row),
        pl.BlockSpec((_BN, _D), row),
    ]
    out_shape = [
        jax.ShapeDtypeStruct((_N, _D), _f32),
        jax.ShapeDtypeStruct((_N, _D), _f32),
    ]
    args = [h, pos128, a0, a1, c0, c1, wh1h, wh1a, bh1, wh2, bh2, g, b]
    if emit_tables:
        in_specs += [pl.BlockSpec((_D, _D), full), pl.BlockSpec((_D, _D), full)]
        out_specs += [pl.BlockSpec((_BN, _D), row), pl.BlockSpec((_BN, _D), row)]
        out_shape += [jax.ShapeDtypeStruct((_N, _D), _f32),
                      jax.ShapeDtypeStruct((_N, _D), _f32)]
        args += [ws, wd]
    return pl.pallas_call(
        _make_node_update_body(emit_tables),
        grid=(nblk,),
        in_specs=in_specs,
        out_specs=out_specs,
        out_shape=out_shape,
    )(*args)


_RC = 128
_NROW = _E // _RC
_RPW = _NROW // _NW
_RREM = _NROW % _NW


def _sc_gather128(ts, td, src, dst):
    mesh = plsc.VectorSubcoreMesh(core_axis_name="c", subcore_axis_name="s")

    @functools.partial(
        pl.kernel,
        mesh=mesh,
        out_type=[
            jax.ShapeDtypeStruct((_E, _D), _f32),
            jax.ShapeDtypeStruct((_E, _D), _f32),
        ],
        scratch_types=[
            pltpu.VMEM((2, 2, _RC), jnp.int32),
            pltpu.VMEM((2, _RC, _D), _f32),
            pltpu.VMEM((2, _RC, _D), _f32),
            pltpu.SemaphoreType.DMA,
            pltpu.SemaphoreType.DMA,
            pltpu.SemaphoreType.DMA,
        ],
        compiler_params=pltpu.CompilerParams(use_tc_tiling_on_sc=True),
    )
    def k(ts_hbm, td_hbm, src_hbm, dst_hbm,
          gs_hbm, gd_hbm,
          idx2, bs2, bd2, semI, semG, semW):
        c = lax.axis_index("c")
        s = lax.axis_index("s")
        w = s * _NC + c
        nr = _RPW + jnp.where(w < _RREM, 1, 0)
        base_row = _RPW * w + jnp.minimum(w, _RREM)

        def start_idx(i, p):
            off = pl.multiple_of((base_row + i) * _RC, _RC)
            pltpu.async_copy(src_hbm.at[pl.ds(off, _RC)], idx2.at[p, 0], semI)
            pltpu.async_copy(dst_hbm.at[pl.ds(off, _RC)], idx2.at[p, 1], semI)

        def wait_idx(p):
            pltpu.make_async_copy(src_hbm.at[pl.ds(0, _RC)],
                                  idx2.at[p, 0], semI).wait()
            pltpu.make_async_copy(dst_hbm.at[pl.ds(0, _RC)],
                                  idx2.at[p, 1], semI).wait()

        def start_gather(p):
            pltpu.async_copy(ts_hbm.at[idx2.at[p, 0]], bs2.at[p], semG)
            pltpu.async_copy(td_hbm.at[idx2.at[p, 1]], bd2.at[p], semG)

        def wait_gather(p):
            pltpu.make_async_copy(ts_hbm.at[idx2.at[p, 0]],
                                  bs2.at[p], semG).wait()
            pltpu.make_async_copy(td_hbm.at[idx2.at[p, 1]],
                                  bd2.at[p], semG).wait()

        def start_wb(i, p):
            off = pl.multiple_of((base_row + i) * _RC, _RC)
            pltpu.async_copy(bs2.at[p], gs_hbm.at[pl.ds(off, _RC)], semW)
            pltpu.async_copy(bd2.at[p], gd_hbm.at[pl.ds(off, _RC)], semW)

        def wait_wb(p):
            pltpu.make_async_copy(bs2.at[p], gs_hbm.at[pl.ds(0, _RC)],
                                  semW).wait()
            pltpu.make_async_copy(bd2.at[p], gd_hbm.at[pl.ds(0, _RC)],
                                  semW).wait()

        start_idx(0, 0)

        def body(i, carry):
            p = lax.rem(i, 2)
            q = 1 - p

            @pl.when(i >= 2)
            def _():
                wait_wb(p)

            @pl.when(i >= 1)
            def _():
                wait_gather(q)
                start_wb(i - 1, q)

            wait_idx(p)
            start_gather(p)

            @pl.when(i + 1 < nr)
            def _():
                start_idx(i + 1, q)

            return carry

        lax.fori_loop(0, nr, body, 0)

        last = lax.rem(nr - 1, 2)
        wait_gather(last)
        start_wb(nr - 1, last)
        wait_wb(1 - last)
        wait_wb(last)

    return k(ts, td, src, dst)


def _sc_scatter128(vals, dst, zeros):
    mesh = plsc.VectorSubcoreMesh(core_axis_name="c", subcore_axis_name="s")

    @functools.partial(
        pl.kernel,
        mesh=mesh,
        out_type=jax.ShapeDtypeStruct((_NC, _N, _D), _f32),
        scratch_types=[
            pltpu.VMEM((2, 1, _RC), jnp.int32),
            pltpu.VMEM((2, _RC, _D), _f32),
            pltpu.VMEM_SHARED((_N, _D), _f32),
            pltpu.SemaphoreType.DMA,
            pltpu.SemaphoreType.DMA,
        ],
        compiler_params=pltpu.CompilerParams(use_tc_tiling_on_sc=True),
    )
    def k(v_hbm, dst_hbm, z_hbm, out_hbm, idx2, vb2, sh, semL, semA):
        c = lax.axis_index("c")
        s = lax.axis_index("s")
        w = c * _NS + s
        nr = _RPW + jnp.where(w < _RREM, 1, 0)
        base_row = _RPW * w + jnp.minimum(w, _RREM)

        @pl.when(s == 0)
        def _():
            pltpu.sync_copy(z_hbm, sh)

        plsc.subcore_barrier()

        def start_loads(i, p):
            off = pl.multiple_of((base_row + i) * _RC, _RC)
            pltpu.async_copy(dst_hbm.at[pl.ds(off, _RC)], idx2.at[p, 0], semL)
            pltpu.async_copy(v_hbm.at[pl.ds(off, _RC)], vb2.at[p], semL)

        def wait_loads(p):
            pltpu.make_async_copy(dst_hbm.at[pl.ds(0, _RC)],
                                  idx2.at[p, 0], semL).wait()
            pltpu.make_async_copy(v_hbm.at[pl.ds(0, _RC)],
                                  vb2.at[p], semL).wait()

        def wait_add(p):
            pltpu.make_async_copy(vb2.at[p], sh.at[idx2.at[p, 0]],
                                  semA).wait()

        start_loads(0, 0)

        def body(i, carry):
            p = lax.rem(i, 2)
            q = 1 - p

            @pl.when(i >= 1)
            def _():
                wait_add(q)

            wait_loads(p)
            pltpu.async_copy(vb2.at[p], sh.at[idx2.at[p, 0]], semA, add=True)

            @pl.when(i + 1 < nr)
            def _():
                start_loads(i + 1, q)

            return carry

        lax.fori_loop(0, nr, body, 0)
        wait_add(lax.rem(nr - 1, 2))

        plsc.subcore_barrier()

        @pl.when(s == 0)
        def _():
            pltpu.sync_copy(sh, out_hbm.at[c])

    return k(vals, dst, zeros)


def kernel(node_feat, edge_attr, pos, Wn, bn, We, be, We1, be1, We2, be2,
           Wx1, bx1, Wx2, bx2, Wh1, bh1, Wh2, bh2, ln_g, ln_b, edge_index):
    src = edge_index[0]
    dst = edge_index[1]

    nf8 = jnp.concatenate(
        [node_feat[:, :6], node_feat[:, 6:7] / _RES_SCALE,
         jnp.zeros((_N, 1), _f32)], axis=1)
    Wn8 = jnp.concatenate([Wn, jnp.zeros((1, _D), _f32)], axis=0)
    pos128 = jnp.concatenate([pos, jnp.zeros((_N, _D - 3), _f32)], axis=1)
    z128 = jnp.zeros((_N, _D), _f32)

    h, ts, td = _node_embed(nf8, Wn8, bn.reshape(1, _D),
                            We1[0, _D:2 * _D], We1[0, 0:_D])
    e = _edge_embed(edge_attr, We, be.reshape(1, _D))

    for l in range(_L):
        w1d2 = We1[l, 2 * _D:2 * _D + 1]
        w1e = We1[l, 2 * _D + 1:]

        gs, gd = _sc_gather128(ts, td, src, dst)
        ps, pd = _sc_gather128(pos128, pos128, src, dst)
        m, wrel = _edge_mlp(gs, gd, e, ps, pd,
                            w1e, w1d2, be1[l].reshape(1, _D),
                            We2[l], be2[l].reshape(1, _D),
                            Wx1[l], bx1[l].reshape(1, _D),
                            Wx2[l].reshape(1, _D), bx2[l].reshape(1, 1))
        aggp = _sc_scatter128(m, dst, z128)
        crdp = _sc_scatter128(wrel, dst, z128)
        if l + 1 < _L:
            h, pos128, ts, td = _node_update(
                h, pos128, aggp[0], aggp[1], crdp[0], crdp[1],
                Wh1[l, :_D], Wh1[l, _D:], bh1[l].reshape(1, _D),
                Wh2[l], bh2[l].reshape(1, _D),
                ln_g[l].reshape(1, _D), ln_b[l].reshape(1, _D),
                We1[l + 1, _D:2 * _D], We1[l + 1, 0:_D])
        else:
            h, pos128 = _node_update(
                h, pos128, aggp[0], aggp[1], crdp[0], crdp[1],
                Wh1[l, :_D], Wh1[l, _D:], bh1[l].reshape(1, _D),
                Wh2[l], bh2[l].reshape(1, _D),
                ln_g[l].reshape(1, _D), ln_b[l].reshape(1, _D))

    return h, pos128[:, :3]


# speedup vs baseline: 1.8046x; 1.1846x over previous
import functools

import jax
import jax.numpy as jnp
from jax import lax
from jax.experimental import pallas as pl
from jax.experimental.pallas import tpu as pltpu
from jax.experimental.pallas import tpu_sc as plsc

_N = 10000
_E = 320000
_D = 128
_L = 3
_NRBF = 32
_RBF_MAX = 10.0
_RES_SCALE = 1000.0
_PW = 16

_NC = 2
_NS = 16
_NW = _NC * _NS
_EPW = _E // _NW
_CHUNK = 200
_NLOOP = _EPW // _CHUNK

_BE_EMB = 3200
_BE = 3200
_BN = 2000

_f32 = jnp.float32


def _silu(x):
    return x / (1.0 + jnp.exp(-x))


def _node_embed_body(nf_ref, w_ref, b_ref, ws_ref, wd_ref,
                     h_ref, ts_ref, td_ref):
    h = jnp.dot(nf_ref[...], w_ref[...], preferred_element_type=_f32) + b_ref[...]
    h_ref[...] = h
    ts_ref[...] = jnp.dot(h, ws_ref[...], preferred_element_type=_f32)
    td_ref[...] = jnp.dot(h, wd_ref[...], preferred_element_type=_f32)


def _node_embed(nf8, Wn8, bn, w1hs, w1hd):
    return pl.pallas_call(
        _node_embed_body,
        out_shape=[
            jax.ShapeDtypeStruct((_N, _D), _f32),
            jax.ShapeDtypeStruct((_N, _D), _f32),
            jax.ShapeDtypeStruct((_N, _D), _f32),
        ],
    )(nf8, Wn8, bn, w1hs, w1hd)


def _rbf_body(ea_ref, o_ref):
    d = ea_ref[...]
    cen = lax.broadcasted_iota(jnp.int32, (_NRBF, 1), 0).astype(_f32) * (
        _RBF_MAX / (_NRBF - 1))
    gamma = 1.0 / ((_RBF_MAX / _NRBF) ** 2)
    o_ref[...] = jnp.exp(-gamma * (d - cen) ** 2)


def _rbf_embed(eaT):
    nblk = _E // _BE_EMB
    return pl.pallas_call(
        _rbf_body,
        grid=(nblk,),
        in_specs=[pl.BlockSpec((1, _BE_EMB), lambda i: (0, i))],
        out_specs=pl.BlockSpec((_NRBF, _BE_EMB), lambda i: (0, i)),
        out_shape=jax.ShapeDtypeStruct((_NRBF, _E), _f32),
    )(eaT)


def _edge_mlp_body(gs_ref, gd_ref, e_ref, ps_ref, pd_ref,
                   w1e_ref, w1d2_ref, b1_ref,
                   w2_ref, b2_ref, wx1_ref, bx1_ref, wx2_ref, bx2_ref,
                   m_ref, wrel_ref):
    rel = pd_ref[...] - ps_ref[...]
    d2 = jnp.sum(rel * rel, axis=1, keepdims=True)
    eterm = lax.dot_general(e_ref[...], w1e_ref[...],
                            (((0,), (0,)), ((), ())),
                            preferred_element_type=_f32)
    z = (gd_ref[...] + gs_ref[...] + eterm
         + d2 * w1d2_ref[...] + b1_ref[...])
    m1 = _silu(z)
    m = _silu(jnp.dot(m1, w2_ref[...], preferred_element_type=_f32) + b2_ref[...])
    t = _silu(jnp.dot(m, wx1_ref[...], preferred_element_type=_f32) + bx1_ref[...])
    w = jnp.sum(t * wx2_ref[...], axis=1, keepdims=True) + bx2_ref[...]
    m_ref[...] = m
    ones_col = (lax.broadcasted_iota(jnp.int32, (1, _D), 1) == 3).astype(_f32)
    wrel_ref[...] = rel * w + ones_col


def _edge_mlp(gs, gd, e, ps, pd, w1e, w1d2, b1, w2, b2, wx1, bx1, wx2, bx2):
    nblk = _E // _BE
    row = lambda i: (i, 0)
    full = lambda i: (0, 0)
    return pl.pallas_call(
        _edge_mlp_body,
        grid=(nblk,),
        in_specs=[
            pl.BlockSpec((_BE, _D), row),
            pl.BlockSpec((_BE, _D), row),
            pl.BlockSpec((_NRBF, _BE), lambda i: (0, i)),
            pl.BlockSpec((_BE, _D), row),
            pl.BlockSpec((_BE, _D), row),
            pl.BlockSpec((_NRBF, _D), full),
            pl.BlockSpec((1, _D), full),
            pl.BlockSpec((1, _D), full),
            pl.BlockSpec((_D, _D), full),
            pl.BlockSpec((1, _D), full),
            pl.BlockSpec((_D, _D), full),
            pl.BlockSpec((1, _D), full),
            pl.BlockSpec((1, _D), full),
            pl.BlockSpec((1, 1), full),
        ],
        out_specs=[
            pl.BlockSpec((_BE, _D), row),
            pl.BlockSpec((_BE, _D), row),
        ],
        out_shape=[
            jax.ShapeDtypeStruct((_E, _D), _f32),
            jax.ShapeDtypeStruct((_E, _D), _f32),
        ],
    )(gs, gd, e, ps, pd, w1e, w1d2, b1, w2, b2, wx1, bx1, wx2, bx2)


def _make_node_update_body(emit_tables):
    def body(*refs):
        if emit_tables:
            (h_ref, p_ref, a0_ref, a1_ref, c0_ref, c1_ref,
             wh1h_ref, wh1a_ref, bh1_ref, wh2_ref, bh2_ref, g_ref, b_ref,
             ws_ref, wd_ref,
             ho_ref, po_ref, ts_ref, td_ref) = refs
        else:
            (h_ref, p_ref, a0_ref, a1_ref, c0_ref, c1_ref,
             wh1h_ref, wh1a_ref, bh1_ref, wh2_ref, bh2_ref, g_ref, b_ref,
             ho_ref, po_ref) = refs
        h = h_ref[...]
        agg = a0_ref[...] + a1_ref[...]
        crd = c0_ref[...] + c1_ref[...]
        deg = crd[:, 3:4]
        posmask = (lax.broadcasted_iota(jnp.int32, (1, _D), 1) < 3).astype(_f32)
        po_ref[...] = p_ref[...] + crd * posmask / (deg + 1.0)
        u = _silu(jnp.dot(h, wh1h_ref[...], preferred_element_type=_f32)
                  + jnp.dot(agg, wh1a_ref[...], preferred_element_type=_f32)
                  + bh1_ref[...])
        h2 = h + jnp.dot(u, wh2_ref[...], preferred_element_type=_f32) + bh2_ref[...]
        mu = jnp.mean(h2, axis=1, keepdims=True)
        dc = h2 - mu
        var = jnp.mean(dc * dc, axis=1, keepdims=True)
        hn = dc * lax.rsqrt(var + 1e-5) * g_ref[...] + b_ref[...]
        ho_ref[...] = hn
        if emit_tables:
            ts_ref[...] = jnp.dot(hn, ws_ref[...], preferred_element_type=_f32)
            td_ref[...] = jnp.dot(hn, wd_ref[...], preferred_element_type=_f32)
    return body


def _node_update(h, pos128, a0, a1, c0, c1, wh1h, wh1a, bh1, wh2, bh2, g, b,
                 ws=None, wd=None):
    emit_tables = ws is not None
    nblk = _N // _BN
    row = lambda i: (i, 0)
    full = lambda i: (0, 0)
    in_specs = [
        pl.BlockSpec((_BN, _D), row),
        pl.BlockSpec((_BN, _D), row),
        pl.BlockSpec((_BN, _D), row),
        pl.BlockSpec((_BN, _D), row),
        pl.BlockSpec((_BN, _D), row),
        pl.BlockSpec((_BN, _D), row),
        pl.BlockSpec((_D, _D), full),
        pl.BlockSpec((_D, _D), full),
        pl.BlockSpec((1, _D), full),
        pl.BlockSpec((_D, _D), full),
        pl.BlockSpec((1, _D), full),
        pl.BlockSpec((1, _D), full),
        pl.BlockSpec((1, _D), full),
    ]
    out_specs = [
        pl.BlockSpec((_BN, _D), row),
        pl.BlockSpec((_BN, _D), row),
    ]
    out_shape = [
        jax.ShapeDtypeStruct((_N, _D), _f32),
        jax.ShapeDtypeStruct((_N, _D), _f32),
    ]
    args = [h, pos128, a0, a1, c0, c1, wh1h, wh1a, bh1, wh2, bh2, g, b]
    if emit_tables:
        in_specs += [pl.BlockSpec((_D, _D), full), pl.BlockSpec((_D, _D), full)]
        out_specs += [pl.BlockSpec((_BN, _D), row), pl.BlockSpec((_BN, _D), row)]
        out_shape += [jax.ShapeDtypeStruct((_N, _D), _f32),
                      jax.ShapeDtypeStruct((_N, _D), _f32)]
        args += [ws, wd]
    return pl.pallas_call(
        _make_node_update_body(emit_tables),
        grid=(nblk,),
        in_specs=in_specs,
        out_specs=out_specs,
        out_shape=out_shape,
    )(*args)


_RC = 128
_NROW = _E // _RC
_RPW = _NROW // _NW
_RREM = _NROW % _NW


def _sc_gather128(ts, td, src, dst):
    mesh = plsc.VectorSubcoreMesh(core_axis_name="c", subcore_axis_name="s")

    @functools.partial(
        pl.kernel,
        mesh=mesh,
        out_type=[
            jax.ShapeDtypeStruct((_E, _D), _f32),
            jax.ShapeDtypeStruct((_E, _D), _f32),
        ],
        scratch_types=[
            pltpu.VMEM((2, 2, _RC), jnp.int32),
            pltpu.VMEM((2, _RC, _D), _f32),
            pltpu.VMEM((2, _RC, _D), _f32),
            pltpu.SemaphoreType.DMA,
            pltpu.SemaphoreType.DMA,
            pltpu.SemaphoreType.DMA,
        ],
        compiler_params=pltpu.CompilerParams(use_tc_tiling_on_sc=True),
    )
    def k(ts_hbm, td_hbm, src_hbm, dst_hbm,
          gs_hbm, gd_hbm,
          idx2, bs2, bd2, semI, semG, semW):
        c = lax.axis_index("c")
        s = lax.axis_index("s")
        w = s * _NC + c
        nr = _RPW + jnp.where(w < _RREM, 1, 0)
        base_row = _RPW * w + jnp.minimum(w, _RREM)

        def start_idx(i, p):
            off = pl.multiple_of((base_row + i) * _RC, _RC)
            pltpu.async_copy(src_hbm.at[pl.ds(off, _RC)], idx2.at[p, 0], semI)
            pltpu.async_copy(dst_hbm.at[pl.ds(off, _RC)], idx2.at[p, 1], semI)

        def wait_idx(p):
            pltpu.make_async_copy(src_hbm.at[pl.ds(0, _RC)],
                                  idx2.at[p, 0], semI).wait()
            pltpu.make_async_copy(dst_hbm.at[pl.ds(0, _RC)],
                                  idx2.at[p, 1], semI).wait()

        def start_gather(p):
            pltpu.async_copy(ts_hbm.at[idx2.at[p, 0]], bs2.at[p], semG)
            pltpu.async_copy(td_hbm.at[idx2.at[p, 1]], bd2.at[p], semG)

        def wait_gather(p):
            pltpu.make_async_copy(ts_hbm.at[idx2.at[p, 0]],
                                  bs2.at[p], semG).wait()
            pltpu.make_async_copy(td_hbm.at[idx2.at[p, 1]],
                                  bd2.at[p], semG).wait()

        def start_wb(i, p):
            off = pl.multiple_of((base_row + i) * _RC, _RC)
            pltpu.async_copy(bs2.at[p], gs_hbm.at[pl.ds(off, _RC)], semW)
            pltpu.async_copy(bd2.at[p], gd_hbm.at[pl.ds(off, _RC)], semW)

        def wait_wb(p):
            pltpu.make_async_copy(bs2.at[p], gs_hbm.at[pl.ds(0, _RC)],
                                  semW).wait()
            pltpu.make_async_copy(bd2.at[p], gd_hbm.at[pl.ds(0, _RC)],
                                  semW).wait()

        start_idx(0, 0)

        def body(i, carry):
            p = lax.rem(i, 2)
            q = 1 - p

            @pl.when(i >= 2)
            def _():
                wait_wb(p)

            @pl.when(i >= 1)
            def _():
                wait_gather(q)
                start_wb(i - 1, q)

            wait_idx(p)
            start_gather(p)

            @pl.when(i + 1 < nr)
            def _():
                start_idx(i + 1, q)

            return carry

        lax.fori_loop(0, nr, body, 0)

        last = lax.rem(nr - 1, 2)
        wait_gather(last)
        start_wb(nr - 1, last)
        wait_wb(1 - last)
        wait_wb(last)

    return k(ts, td, src, dst)


def _sc_scatter128(vals, dst, zeros):
    mesh = plsc.VectorSubcoreMesh(core_axis_name="c", subcore_axis_name="s")

    @functools.partial(
        pl.kernel,
        mesh=mesh,
        out_type=jax.ShapeDtypeStruct((_NC, _N, _D), _f32),
        scratch_types=[
            pltpu.VMEM((2, 1, _RC), jnp.int32),
            pltpu.VMEM((2, _RC, _D), _f32),
            pltpu.VMEM_SHARED((_N, _D), _f32),
            pltpu.SemaphoreType.DMA,
            pltpu.SemaphoreType.DMA,
        ],
        compiler_params=pltpu.CompilerParams(use_tc_tiling_on_sc=True),
    )
    def k(v_hbm, dst_hbm, z_hbm, out_hbm, idx2, vb2, sh, semL, semA):
        c = lax.axis_index("c")
        s = lax.axis_index("s")
        w = c * _NS + s
        nr = _RPW + jnp.where(w < _RREM, 1, 0)
        base_row = _RPW * w + jnp.minimum(w, _RREM)

        @pl.when(s == 0)
        def _():
            pltpu.sync_copy(z_hbm, sh)

        plsc.subcore_barrier()

        def start_loads(i, p):
            off = pl.multiple_of((base_row + i) * _RC, _RC)
            pltpu.async_copy(dst_hbm.at[pl.ds(off, _RC)], idx2.at[p, 0], semL)
            pltpu.async_copy(v_hbm.at[pl.ds(off, _RC)], vb2.at[p], semL)

        def wait_loads(p):
            pltpu.make_async_copy(dst_hbm.at[pl.ds(0, _RC)],
                                  idx2.at[p, 0], semL).wait()
            pltpu.make_async_copy(v_hbm.at[pl.ds(0, _RC)],
                                  vb2.at[p], semL).wait()

        def wait_add(p):
            pltpu.make_async_copy(vb2.at[p], sh.at[idx2.at[p, 0]],
                                  semA).wait()

        start_loads(0, 0)

        def body(i, carry):
            p = lax.rem(i, 2)
            q = 1 - p

            @pl.when(i >= 1)
            def _():
                wait_add(q)

            wait_loads(p)
            pltpu.async_copy(vb2.at[p], sh.at[idx2.at[p, 0]], semA, add=True)

            @pl.when(i + 1 < nr)
            def _():
                start_loads(i + 1, q)

            return carry

        lax.fori_loop(0, nr, body, 0)
        wait_add(lax.rem(nr - 1, 2))

        plsc.subcore_barrier()

        @pl.when(s == 0)
        def _():
            pltpu.sync_copy(sh, out_hbm.at[c])

    return k(vals, dst, zeros)


def kernel(node_feat, edge_attr, pos, Wn, bn, We, be, We1, be1, We2, be2,
           Wx1, bx1, Wx2, bx2, Wh1, bh1, Wh2, bh2, ln_g, ln_b, edge_index):
    src = edge_index[0]
    dst = edge_index[1]

    nf8 = jnp.concatenate(
        [node_feat[:, :6], node_feat[:, 6:7] / _RES_SCALE,
         jnp.zeros((_N, 1), _f32)], axis=1)
    Wn8 = jnp.concatenate([Wn, jnp.zeros((1, _D), _f32)], axis=0)
    pos128 = jnp.concatenate([pos, jnp.zeros((_N, _D - 3), _f32)], axis=1)
    z128 = jnp.zeros((_N, _D), _f32)

    h, ts, td = _node_embed(nf8, Wn8, bn.reshape(1, _D),
                            We1[0, _D:2 * _D], We1[0, 0:_D])
    rbf = _rbf_embed(edge_attr.reshape(1, _E))

    for l in range(_L):
        w1d2 = We1[l, 2 * _D:2 * _D + 1]
        w1e = We1[l, 2 * _D + 1:]
        v_l = We @ w1e
        b1_l = (be1[l] + be @ w1e).reshape(1, _D)

        gs, gd = _sc_gather128(ts, td, src, dst)
        ps, pd = _sc_gather128(pos128, pos128, src, dst)
        m, wrel = _edge_mlp(gs, gd, rbf, ps, pd,
                            v_l, w1d2, b1_l,
                            We2[l], be2[l].reshape(1, _D),
                            Wx1[l], bx1[l].reshape(1, _D),
                            Wx2[l].reshape(1, _D), bx2[l].reshape(1, 1))
        aggp = _sc_scatter128(m, dst, z128)
        crdp = _sc_scatter128(wrel, dst, z128)
        if l + 1 < _L:
            h, pos128, ts, td = _node_update(
                h, pos128, aggp[0], aggp[1], crdp[0], crdp[1],
                Wh1[l, :_D], Wh1[l, _D:], bh1[l].reshape(1, _D),
                Wh2[l], bh2[l].reshape(1, _D),
                ln_g[l].reshape(1, _D), ln_b[l].reshape(1, _D),
                We1[l + 1, _D:2 * _D], We1[l + 1, 0:_D])
        else:
            h, pos128 = _node_update(
                h, pos128, aggp[0], aggp[1], crdp[0], crdp[1],
                Wh1[l, :_D], Wh1[l, _D:], bh1[l].reshape(1, _D),
                Wh2[l], bh2[l].reshape(1, _D),
                ln_g[l].reshape(1, _D), ln_b[l].reshape(1, _D))

    return h, pos128[:, :3]
